# baseline (speedup 1.0000x reference)
import functools

import jax
import jax.numpy as jnp
import numpy as np
from jax import lax
from jax.experimental import pallas as pl
from jax.experimental.pallas import tpu as pltpu
from jax.experimental.pallas import tpu_sc as plsc

IN_CH = 768
K = 9
GROUPS = 32
H = W = 16
N = H * W
B = 16
NUM_NODES = B * N
NW = 32
NODES_PER_W = NUM_NODES // NW
CHUNK = 8

_F32_MIN = -3.0e38


def _sincos_1d(embed_dim, pos):
    omega = np.arange(embed_dim // 2, dtype=np.float64)
    omega = omega / (embed_dim / 2.0)
    omega = 1.0 / (10000.0 ** omega)
    out = np.einsum('m,d->md', pos.reshape(-1).astype(np.float64), omega)
    return np.concatenate([np.sin(out), np.cos(out)], axis=1)


def _relative_pos_np(embed_dim, grid_size):
    gh = np.arange(grid_size, dtype=np.float32)
    gw = np.arange(grid_size, dtype=np.float32)
    grid = np.stack(np.meshgrid(gw, gh), axis=0)
    emb_h = _sincos_1d(embed_dim // 2, grid[0])
    emb_w = _sincos_1d(embed_dim // 2, grid[1])
    pe = np.concatenate([emb_h, emb_w], axis=1)
    rp = 2.0 * (pe @ pe.T) / pe.shape[1]
    return rp.astype(np.float32)


def _group_map(channels, groups):
    m = np.zeros((channels, groups), dtype=np.float32)
    per = channels // groups
    for g in range(groups):
        m[g * per:(g + 1) * per, g] = 1.0
    return m


def _dot(a, b, precision=jax.lax.Precision.HIGHEST):
    return jax.lax.dot_general(a, b, (((1,), (0,)), ((), ())),
                               preferred_element_type=jnp.float32,
                               precision=precision)


def _group_norm(y, gmap, gmapT, gg, gb, nelem):
    s = jnp.sum(y, axis=0, keepdims=True)
    ss = jnp.sum(y * y, axis=0, keepdims=True)
    gs = _dot(s, gmap)
    gss = _dot(ss, gmap)
    mu = gs / nelem
    var = gss / nelem - mu * mu
    rstd = jax.lax.rsqrt(var + 1e-5)
    mu_c = _dot(mu, gmapT)
    rstd_c = _dot(rstd, gmapT)
    return (y - mu_c) * rstd_c * gg + gb


def _body_a(x_ref, fc1_wT_ref, fc1_b_ref, fc1_gg_ref, fc1_gb_ref,
            rel_pos_ref, gmapC_ref, gmapCT_ref,
            xf_ref, idx_ref):
    b = pl.program_id(0)
    x = x_ref[0]
    y = _dot(x, fc1_wT_ref[...], precision=None) + fc1_b_ref[...]
    xf = _group_norm(y, gmapC_ref[...], gmapCT_ref[...],
                     fc1_gg_ref[...], fc1_gb_ref[...],
                     float((IN_CH // GROUPS) * N))
    xf_ref[0] = xf

    nrm = jnp.sqrt(jnp.sum(xf * xf, axis=1, keepdims=True))
    xn = xf / jnp.maximum(nrm, 1e-12)
    inner = jax.lax.dot_general(xn, xn, (((1,), (1,)), ((), ())),
                                preferred_element_type=jnp.float32,
                                precision=None)
    sq = jnp.sum(xn * xn, axis=1, keepdims=True)
    dist = 2.0 * inner - sq - jnp.transpose(sq) + rel_pos_ref[...]

    cols = jax.lax.broadcasted_iota(jnp.int32, (N, N), 1)
    d = dist
    picks = []
    for _ in range(K):
        m = jnp.max(d, axis=1, keepdims=True)
        amin = jnp.min(jnp.where(d == m, cols, N), axis=1, keepdims=True)
        picks.append(amin + b * N)
        d = jnp.where(cols == amin, _F32_MIN, d)
    idx_ref[0] = jnp.concatenate(picks, axis=1)


@jax.jit
def _run_a(xf_in, fc1_wT, fc1_b, fc1_gg, fc1_gb, rel_pos, gmapC, gmapCT):
    C = IN_CH
    nb = xf_in.shape[0]
    row = lambda c: pl.BlockSpec((1, c), lambda b: (0, 0))
    full = lambda r, c: pl.BlockSpec((r, c), lambda b: (0, 0))
    return pl.pallas_call(
        _body_a,
        grid=(nb,),
        in_specs=[
            pl.BlockSpec((1, N, C), lambda b: (b, 0, 0)),
            full(C, C), row(C), row(C), row(C),
            full(N, N), full(C, GROUPS), full(GROUPS, C),
        ],
        out_specs=[
            pl.BlockSpec((1, N, C), lambda b: (b, 0, 0)),
            pl.BlockSpec((1, N, K), lambda b: (b, 0, 0)),
        ],
        out_shape=[
            jax.ShapeDtypeStruct((nb, N, C), jnp.float32),
            jax.ShapeDtypeStruct((nb, N, K), jnp.int32),
        ],
        compiler_params=pltpu.CompilerParams(
            dimension_semantics=("arbitrary",),
        ),
    )(xf_in, fc1_wT, fc1_b, fc1_gg, fc1_gb, rel_pos, gmapC, gmapCT)


def _sc_gather_max(xf_flat, idx_flat):
    mesh = plsc.VectorSubcoreMesh(core_axis_name="c", subcore_axis_name="s")
    num_nodes = xf_flat.shape[0]
    nodes_per_w = num_nodes // NW
    nchunks = nodes_per_w // CHUNK
    CW = IN_CH

    @functools.partial(
        pl.kernel, mesh=mesh,
        out_type=jax.ShapeDtypeStruct((num_nodes, CW), jnp.float32),
        scratch_types=[
            pltpu.VMEM((2, CHUNK * K), jnp.int32),
            pltpu.VMEM((2, CHUNK * K, CW), jnp.float32),
            pltpu.VMEM((CHUNK, CW), jnp.float32),
            pltpu.SemaphoreType.DMA,
            pltpu.SemaphoreType.DMA,
        ],
    )
    def k(xf_hbm, idx_hbm, out_hbm, idx_v, rows_v, out_v, sem0, sem1):
        wid = lax.axis_index("s") * 2 + lax.axis_index("c")
        node0 = wid * nodes_per_w
        sems = (sem0, sem1)

        pltpu.sync_copy(idx_hbm.at[pl.ds(node0 * K, CHUNK * K)], idx_v.at[0])
        handles = {0: pltpu.async_copy(xf_hbm.at[idx_v.at[0]],
                                       rows_v.at[0], sems[0])}
        for c in range(nchunks):
            cur = c % 2
            nxt = (c + 1) % 2
            if c + 1 < nchunks:
                nbase = node0 + (c + 1) * CHUNK
                pltpu.sync_copy(idx_hbm.at[pl.ds(nbase * K, CHUNK * K)],
                                idx_v.at[nxt])
                handles[c + 1] = pltpu.async_copy(
                    xf_hbm.at[idx_v.at[nxt]], rows_v.at[nxt], sems[nxt])
            handles[c].wait()

            def cb_body(j, carry, cur=cur):
                off = j * 16
                for n in range(CHUNK):
                    r = n * K
                    acc = rows_v[cur, r, pl.ds(off, 16)]
                    for kk in range(1, K):
                        acc = jnp.maximum(acc, rows_v[cur, r + kk, pl.ds(off, 16)])
                    out_v[n, pl.ds(off, 16)] = acc
                return carry

            lax.fori_loop(0, CW // 16, cb_body, 0)
            pltpu.sync_copy(out_v, out_hbm.at[pl.ds(node0 + c * CHUNK, CHUNK)])

    return k(xf_flat, idx_flat)


def _body_b(x_ref, xf_ref, acc_ref,
            gw1_ref, gw2_ref, g_b_ref, g_gg_ref, g_gb_ref,
            fc2_wT_ref, fc2_b_ref, fc2_gg_ref, fc2_gb_ref,
            gmapC_ref, gmapCT_ref, gmap2C_ref, gmap2CT_ref,
            out_ref):
    x = x_ref[0]
    xf = xf_ref[0]
    x_j = acc_ref[0].astype(jnp.float32) - xf

    t = (_dot(xf.astype(jnp.bfloat16), gw1_ref[...], precision=None)
         + _dot(x_j.astype(jnp.bfloat16), gw2_ref[...], precision=None)
         + g_b_ref[...])
    t = _group_norm(t, gmap2C_ref[...], gmap2CT_ref[...],
                    g_gg_ref[...], g_gb_ref[...],
                    float((2 * IN_CH // GROUPS) * N))
    u = 0.5 * t * (1.0 + jax.lax.erf(t * np.float32(1.0 / np.sqrt(2.0))))

    z = (_dot(u.astype(jnp.bfloat16), fc2_wT_ref[...], precision=None)
         + fc2_b_ref[...])
    z = _group_norm(z, gmapC_ref[...], gmapCT_ref[...],
                    fc2_gg_ref[...], fc2_gb_ref[...],
                    float((IN_CH // GROUPS) * N))
    out_ref[0] = z + x


@jax.jit
def _run_b(xf_in, xf, acc, gw1, gw2, g_b, g_gg, g_gb,
           fc2_wT, fc2_b, fc2_gg, fc2_gb, gmapC, gmapCT, gmap2C, gmap2CT):
    C = IN_CH
    nb = xf_in.shape[0]
    row = lambda c: pl.BlockSpec((1, c), lambda b: (0, 0))
    full = lambda r, c: pl.BlockSpec((r, c), lambda b: (0, 0))
    img = pl.BlockSpec((1, N, C), lambda b: (b, 0, 0))
    return pl.pallas_call(
        _body_b,
        grid=(nb,),
        in_specs=[
            img, img, img,
            full(C, 2 * C), full(C, 2 * C), row(2 * C), row(2 * C), row(2 * C),
            full(2 * C, C), row(C), row(C), row(C),
            full(C, GROUPS), full(GROUPS, C),
            full(2 * C, GROUPS), full(GROUPS, 2 * C),
        ],
        out_specs=img,
        out_shape=jax.ShapeDtypeStruct((nb, N, C), jnp.float32),
        compiler_params=pltpu.CompilerParams(
            dimension_semantics=("arbitrary",),
        ),
    )(xf_in, xf, acc, gw1, gw2, g_b, g_gg, g_gb,
      fc2_wT, fc2_b, fc2_gg, fc2_gb, gmapC, gmapCT, gmap2C, gmap2CT)


def kernel(x, fc1_w, fc1_b, fc1_gg, fc1_gb, g_w, g_b, g_gg, g_gb,
           fc2_w, fc2_b, fc2_gg, fc2_gb):
    Bx, C, Hx, Wx = x.shape
    xf_in = jnp.transpose(x.reshape(Bx, C, N), (0, 2, 1))
    rel_pos = jnp.asarray(_relative_pos_np(C, Hx))
    gmapC = jnp.asarray(_group_map(C, GROUPS))
    gmap2C = jnp.asarray(_group_map(2 * C, GROUPS))
    gw = jnp.transpose(g_w).astype(jnp.bfloat16)

    S = 4
    gb = B // S
    xfs, idxs, accs, outs = [], [], [], []
    for g in range(S):
        xf_g, idx_g = _run_a(xf_in[g * gb:(g + 1) * gb],
                             jnp.transpose(fc1_w), fc1_b[None, :],
                             fc1_gg[None, :], fc1_gb[None, :], rel_pos,
                             gmapC, jnp.transpose(gmapC))
        xfs.append(xf_g)
        idxs.append(idx_g)
    for g in range(S):
        accs.append(_sc_gather_max(xfs[g].reshape(gb * N, C),
                                   idxs[g].reshape(-1)))
    for g in range(S):
        outs.append(_run_b(
            xf_in[g * gb:(g + 1) * gb], xfs[g], accs[g].reshape(gb, N, C),
            gw[:C], gw[C:], g_b[None, :], g_gg[None, :], g_gb[None, :],
            jnp.transpose(fc2_w).astype(jnp.bfloat16), fc2_b[None, :],
            fc2_gg[None, :], fc2_gb[None, :],
            gmapC, jnp.transpose(gmapC), gmap2C, jnp.transpose(gmap2C)))
    out = jnp.concatenate(outs, axis=0)
    return jnp.transpose(out, (0, 2, 1)).reshape(Bx, C, Hx, Wx)


# speedup vs baseline: 1.1823x; 1.1823x over previous
import functools

import jax
import jax.numpy as jnp
import numpy as np
from jax.experimental import pallas as pl
from jax.experimental.pallas import tpu as pltpu

IN_CH = 768
K = 9
GROUPS = 32
H = W = 16
N = H * W
B = 16

_F32_MIN = -3.0e38


def _sincos_1d(embed_dim, pos):
    omega = np.arange(embed_dim // 2, dtype=np.float64)
    omega = omega / (embed_dim / 2.0)
    omega = 1.0 / (10000.0 ** omega)
    out = np.einsum('m,d->md', pos.reshape(-1).astype(np.float64), omega)
    return np.concatenate([np.sin(out), np.cos(out)], axis=1)


def _relative_pos_np(embed_dim, grid_size):
    gh = np.arange(grid_size, dtype=np.float32)
    gw = np.arange(grid_size, dtype=np.float32)
    grid = np.stack(np.meshgrid(gw, gh), axis=0)
    emb_h = _sincos_1d(embed_dim // 2, grid[0])
    emb_w = _sincos_1d(embed_dim // 2, grid[1])
    pe = np.concatenate([emb_h, emb_w], axis=1)
    rp = 2.0 * (pe @ pe.T) / pe.shape[1]
    return rp.astype(np.float32)


def _group_map(channels, groups):
    m = np.zeros((channels, groups), dtype=np.float32)
    per = channels // groups
    for g in range(groups):
        m[g * per:(g + 1) * per, g] = 1.0
    return m


def _dot(a, b, precision=jax.lax.Precision.HIGHEST):
    return jax.lax.dot_general(a, b, (((1,), (0,)), ((), ())),
                               preferred_element_type=jnp.float32,
                               precision=precision)


def _group_norm(y, gmap, gmapT, gg, gb, nelem):
    ones_r = jnp.ones((1, y.shape[0]), dtype=jnp.float32)
    s = _dot(ones_r, y)
    ss = _dot(ones_r, y * y)
    gs = _dot(s, gmap)
    gss = _dot(ss, gmap)
    mu = gs / nelem
    var = gss / nelem - mu * mu
    rstd = jax.lax.rsqrt(var + 1e-5)
    mu_c = _dot(mu, gmapT)
    rstd_c = _dot(rstd, gmapT)
    a = rstd_c * gg
    b = gb - mu_c * a
    return y * a + b


def _erf(z):
    return jax.lax.erf(z)


def _body(x_ref, fc1_wT_ref, fc1_b_ref, fc1_gg_ref, fc1_gb_ref,
          gw1_ref, gw2_ref, g_b_ref, g_gg_ref, g_gb_ref,
          fc2_wT_ref, fc2_b_ref, fc2_gg_ref, fc2_gb_ref,
          rel_pos_ref, gmapC_ref, gmapCT_ref, gmap2C_ref, gmap2CT_ref,
          out_ref):
    x = x_ref[0]
    y = _dot(x, fc1_wT_ref[...], precision=None) + fc1_b_ref[...]
    xf = _group_norm(y, gmapC_ref[...], gmapCT_ref[...],
                     fc1_gg_ref[...], fc1_gb_ref[...],
                     float((IN_CH // GROUPS) * N))

    nrm = jnp.sqrt(jnp.sum(xf * xf, axis=1, keepdims=True))
    xn = xf / jnp.maximum(nrm, 1e-12)
    inner = jax.lax.dot_general(xn, xn, (((1,), (1,)), ((), ())),
                                preferred_element_type=jnp.float32,
                                precision=None)
    sq = jnp.sum(xn * xn, axis=1, keepdims=True)
    dist = 2.0 * inner - sq - jnp.transpose(sq) + rel_pos_ref[...]

    xf_bf = xf.astype(jnp.bfloat16)
    cols = jax.lax.broadcasted_iota(jnp.int32, (N, N), 1)
    d = dist
    acc = jnp.full((N, IN_CH), _F32_MIN, dtype=jnp.float32)
    for _ in range(K):
        m = jnp.max(d, axis=1, keepdims=True)
        amin = jnp.min(jnp.where(d == m, cols, N), axis=1, keepdims=True)
        onehot = (cols == amin).astype(jnp.bfloat16)
        acc = jnp.maximum(acc, _dot(onehot, xf_bf, precision=None))
        d = jnp.where(cols == amin, _F32_MIN, d)
    x_j = acc - xf

    t = (_dot(xf_bf, gw1_ref[...], precision=None)
         + _dot(x_j.astype(jnp.bfloat16), gw2_ref[...], precision=None)
         + g_b_ref[...])
    t = _group_norm(t, gmap2C_ref[...], gmap2CT_ref[...],
                    g_gg_ref[...], g_gb_ref[...],
                    float((2 * IN_CH // GROUPS) * N))
    u = 0.5 * t * (1.0 + _erf(t * np.float32(1.0 / np.sqrt(2.0))))

    z = _dot(u.astype(jnp.bfloat16), fc2_wT_ref[...], precision=None) + fc2_b_ref[...]
    z = _group_norm(z, gmapC_ref[...], gmapCT_ref[...],
                    fc2_gg_ref[...], fc2_gb_ref[...],
                    float((IN_CH // GROUPS) * N))
    out_ref[0] = z + x


@functools.partial(jax.jit, static_argnames=())
def _run(xf_in, fc1_wT, fc1_b, fc1_gg, fc1_gb, gw1, gw2, g_b, g_gg, g_gb,
         fc2_wT, fc2_b, fc2_gg, fc2_gb, rel_pos, gmapC, gmapCT, gmap2C, gmap2CT):
    C = IN_CH
    row = lambda c: pl.BlockSpec((1, c), lambda b: (0, 0))
    full = lambda r, c: pl.BlockSpec((r, c), lambda b: (0, 0))
    return pl.pallas_call(
        _body,
        grid=(B,),
        in_specs=[
            pl.BlockSpec((1, N, C), lambda b: (b, 0, 0)),
            full(C, C), row(C), row(C), row(C),
            full(C, 2 * C), full(C, 2 * C), row(2 * C), row(2 * C), row(2 * C),
            full(2 * C, C), row(C), row(C), row(C),
            full(N, N), full(C, GROUPS), full(GROUPS, C),
            full(2 * C, GROUPS), full(GROUPS, 2 * C),
        ],
        out_specs=pl.BlockSpec((1, N, C), lambda b: (b, 0, 0)),
        out_shape=jax.ShapeDtypeStruct((B, N, C), jnp.float32),
        compiler_params=pltpu.CompilerParams(
            dimension_semantics=("arbitrary",),
        ),
    )(xf_in, fc1_wT, fc1_b, fc1_gg, fc1_gb, gw1, gw2, g_b, g_gg, g_gb,
      fc2_wT, fc2_b, fc2_gg, fc2_gb, rel_pos, gmapC, gmapCT, gmap2C, gmap2CT)


def kernel(x, fc1_w, fc1_b, fc1_gg, fc1_gb, g_w, g_b, g_gg, g_gb,
           fc2_w, fc2_b, fc2_gg, fc2_gb):
    Bx, C, Hx, Wx = x.shape
    xf_in = jnp.transpose(x.reshape(Bx, C, N), (0, 2, 1))
    rel_pos = jnp.asarray(_relative_pos_np(C, Hx))
    gmapC = jnp.asarray(_group_map(C, GROUPS))
    gmap2C = jnp.asarray(_group_map(2 * C, GROUPS))
    gw = jnp.transpose(g_w).astype(jnp.bfloat16)
    out = _run(
        xf_in, jnp.transpose(fc1_w), fc1_b[None, :], fc1_gg[None, :],
        fc1_gb[None, :], gw[:C], gw[C:], g_b[None, :], g_gg[None, :],
        g_gb[None, :], jnp.transpose(fc2_w).astype(jnp.bfloat16),
        fc2_b[None, :], fc2_gg[None, :],
        fc2_gb[None, :], rel_pos, gmapC, jnp.transpose(gmapC),
        gmap2C, jnp.transpose(gmap2C),
    )
    return jnp.transpose(out, (0, 2, 1)).reshape(Bx, C, Hx, Wx)


# speedup vs baseline: 1.4569x; 1.2322x over previous
import functools

import jax
import jax.numpy as jnp
import numpy as np
from jax.experimental import pallas as pl
from jax.experimental.pallas import tpu as pltpu

IN_CH = 768
K = 9
GROUPS = 32
H = W = 16
N = H * W
B = 16

_F32_MIN = -3.0e38


def _sincos_1d(embed_dim, pos):
    omega = np.arange(embed_dim // 2, dtype=np.float64)
    omega = omega / (embed_dim / 2.0)
    omega = 1.0 / (10000.0 ** omega)
    out = np.einsum('m,d->md', pos.reshape(-1).astype(np.float64), omega)
    return np.concatenate([np.sin(out), np.cos(out)], axis=1)


def _relative_pos_np(embed_dim, grid_size):
    gh = np.arange(grid_size, dtype=np.float32)
    gw = np.arange(grid_size, dtype=np.float32)
    grid = np.stack(np.meshgrid(gw, gh), axis=0)
    emb_h = _sincos_1d(embed_dim // 2, grid[0])
    emb_w = _sincos_1d(embed_dim // 2, grid[1])
    pe = np.concatenate([emb_h, emb_w], axis=1)
    rp = 2.0 * (pe @ pe.T) / pe.shape[1]
    return rp.astype(np.float32)


def _group_map(channels, groups):
    m = np.zeros((channels, groups), dtype=np.float32)
    per = channels // groups
    for g in range(groups):
        m[g * per:(g + 1) * per, g] = 1.0
    return m


def _dot(a, b, precision=jax.lax.Precision.HIGHEST):
    return jax.lax.dot_general(a, b, (((1,), (0,)), ((), ())),
                               preferred_element_type=jnp.float32,
                               precision=precision)


def _group_norm(y, gmap, gmapT, gg, gb, nelem):
    s = jnp.sum(y, axis=0, keepdims=True)
    ss = jnp.sum(y * y, axis=0, keepdims=True)
    gs = _dot(s, gmap)
    gss = _dot(ss, gmap)
    mu = gs / nelem
    var = gss / nelem - mu * mu
    rstd = jax.lax.rsqrt(var + 1e-5)
    mu_c = _dot(mu, gmapT)
    rstd_c = _dot(rstd, gmapT)
    a = rstd_c * gg
    b = gb - mu_c * a
    return y * a + b


def _erf(z):
    return jax.lax.erf(z)


def _body(x_ref, fc1_wT_ref, fc1_b_ref, fc1_gg_ref, fc1_gb_ref,
          gw1_ref, gw2_ref, g_b_ref, g_gg_ref, g_gb_ref,
          fc2_wT_ref, fc2_b_ref, fc2_gg_ref, fc2_gb_ref,
          rel_pos_ref, gmapC_ref, gmapCT_ref, gmap2C_ref, gmap2CT_ref,
          out_ref):
    x = x_ref[0]
    y = _dot(x, fc1_wT_ref[...], precision=None) + fc1_b_ref[...]
    xf = _group_norm(y, gmapC_ref[...], gmapCT_ref[...],
                     fc1_gg_ref[...], fc1_gb_ref[...],
                     float((IN_CH // GROUPS) * N))

    nrm = jnp.sqrt(jnp.sum(xf * xf, axis=1, keepdims=True))
    xn = xf / jnp.maximum(nrm, 1e-12)
    inner = jax.lax.dot_general(xn, xn, (((1,), (1,)), ((), ())),
                                preferred_element_type=jnp.float32,
                                precision=None)
    sq = jnp.sum(xn * xn, axis=1, keepdims=True)
    dist = 2.0 * inner - sq - jnp.transpose(sq) + rel_pos_ref[...]

    xf_bf = xf.astype(jnp.bfloat16)
    cols = jax.lax.broadcasted_iota(jnp.int32, (N, N), 1)
    d = dist
    acc = jnp.full((N, IN_CH), _F32_MIN, dtype=jnp.float32)
    for _ in range(K):
        m = jnp.max(d, axis=1, keepdims=True)
        amin = jnp.min(jnp.where(d == m, cols, N), axis=1, keepdims=True)
        onehot = (cols == amin).astype(jnp.bfloat16)
        acc = jnp.maximum(acc, _dot(onehot, xf_bf, precision=None))
        d = jnp.where(cols == amin, _F32_MIN, d)
    x_j = acc - xf

    t = (_dot(xf_bf, gw1_ref[...], precision=None)
         + _dot(x_j.astype(jnp.bfloat16), gw2_ref[...], precision=None)
         + g_b_ref[...])
    t = _group_norm(t, gmap2C_ref[...], gmap2CT_ref[...],
                    g_gg_ref[...], g_gb_ref[...],
                    float((2 * IN_CH // GROUPS) * N))
    u = 0.5 * t * (1.0 + _erf(t * np.float32(1.0 / np.sqrt(2.0))))

    z = _dot(u.astype(jnp.bfloat16), fc2_wT_ref[...], precision=None) + fc2_b_ref[...]
    z = _group_norm(z, gmapC_ref[...], gmapCT_ref[...],
                    fc2_gg_ref[...], fc2_gb_ref[...],
                    float((IN_CH // GROUPS) * N))
    out_ref[0] = z + x


@functools.partial(jax.jit, static_argnames=())
def _run(xf_in, fc1_wT, fc1_b, fc1_gg, fc1_gb, gw1, gw2, g_b, g_gg, g_gb,
         fc2_wT, fc2_b, fc2_gg, fc2_gb, rel_pos, gmapC, gmapCT, gmap2C, gmap2CT):
    C = IN_CH
    row = lambda c: pl.BlockSpec((1, c), lambda b: (0, 0))
    full = lambda r, c: pl.BlockSpec((r, c), lambda b: (0, 0))
    return pl.pallas_call(
        _body,
        grid=(B,),
        in_specs=[
            pl.BlockSpec((1, N, C), lambda b: (b, 0, 0)),
            full(C, C), row(C), row(C), row(C),
            full(C, 2 * C), full(C, 2 * C), row(2 * C), row(2 * C), row(2 * C),
            full(2 * C, C), row(C), row(C), row(C),
            full(N, N), full(C, GROUPS), full(GROUPS, C),
            full(2 * C, GROUPS), full(GROUPS, 2 * C),
        ],
        out_specs=pl.BlockSpec((1, N, C), lambda b: (b, 0, 0)),
        out_shape=jax.ShapeDtypeStruct((B, N, C), jnp.float32),
        compiler_params=pltpu.CompilerParams(
            dimension_semantics=("arbitrary",),
        ),
    )(xf_in, fc1_wT, fc1_b, fc1_gg, fc1_gb, gw1, gw2, g_b, g_gg, g_gb,
      fc2_wT, fc2_b, fc2_gg, fc2_gb, rel_pos, gmapC, gmapCT, gmap2C, gmap2CT)


def kernel(x, fc1_w, fc1_b, fc1_gg, fc1_gb, g_w, g_b, g_gg, g_gb,
           fc2_w, fc2_b, fc2_gg, fc2_gb):
    Bx, C, Hx, Wx = x.shape
    xf_in = jnp.transpose(x.reshape(Bx, C, N), (0, 2, 1))
    rel_pos = jnp.asarray(_relative_pos_np(C, Hx))
    gmapC = jnp.asarray(_group_map(C, GROUPS))
    gmap2C = jnp.asarray(_group_map(2 * C, GROUPS))
    gw = jnp.transpose(g_w).astype(jnp.bfloat16)
    out = _run(
        xf_in, jnp.transpose(fc1_w), fc1_b[None, :], fc1_gg[None, :],
        fc1_gb[None, :], gw[:C], gw[C:], g_b[None, :], g_gg[None, :],
        g_gb[None, :], jnp.transpose(fc2_w).astype(jnp.bfloat16),
        fc2_b[None, :], fc2_gg[None, :],
        fc2_gb[None, :], rel_pos, gmapC, jnp.transpose(gmapC),
        gmap2C, jnp.transpose(gmap2C),
    )
    return jnp.transpose(out, (0, 2, 1)).reshape(Bx, C, Hx, Wx)
